# Initial kernel scaffold; baseline (speedup 1.0000x reference)
import jax
import jax.numpy as jnp
from jax.experimental import pallas as pl


def kernel(input_data, knn, We1, be1, Wg1, bg1, Wg2, bg2, We2, be2, Wd1, bd1, Wd2, bd2):
    raise NotImplementedError("write your pallas kernel here")


# speedup vs baseline: 7.5744x; 7.5744x over previous
import functools

import jax
import jax.numpy as jnp
import numpy as np
from jax import lax
from jax.experimental import pallas as pl
from jax.experimental.pallas import tpu as pltpu
from jax.experimental.pallas import tpu_sc as plsc

F32 = jnp.float32
I32 = jnp.int32

H1 = 256
H3 = 128
B = 4
N = 2500
E = 160000
BN = B * N
R = 10240
EP = 163840
NCH = EP // 128
RPT = R // 16

_SELU_A = 1.6732632423543772
_SELU_S = 1.0507009873554805


def _selu(x):
    return _SELU_S * jnp.where(x > 0, x, _SELU_A * (jnp.exp(x) - 1.0))


def _mesh():
    return plsc.VectorSubcoreMesh(core_axis_name="c", subcore_axis_name="s")


def _deg_body(dst_hbm, out_hbm, dstb, onesb, acc, sem):
    c = lax.axis_index("c")
    s = lax.axis_index("s")
    wid = s * 2 + c
    npc = NCH // 32
    pltpu.sync_copy(dst_hbm.at[pl.ds(wid * npc, npc)], dstb)
    zero16 = jnp.zeros((16,), F32)
    ones16 = jnp.ones((16,), F32)

    def zbody(i, carry):
        onesb[i, pl.ds(0, 16)] = zero16
        return carry

    lax.fori_loop(0, 128, zbody, 0)
    base = s * RPT
    for k in range(RPT // 128):
        pltpu.sync_copy(onesb, acc.at[pl.ds(base + k * 128, 128)])

    def obody(i, carry):
        onesb[i, pl.ds(0, 16)] = ones16
        return carry

    lax.fori_loop(0, 128, obody, 0)
    plsc.subcore_barrier()

    def chunk(j, carry):
        pltpu.async_copy(onesb, acc.at[dstb.at[j]], sem, add=True).wait()
        return carry

    lax.fori_loop(0, npc, chunk, 0)
    plsc.subcore_barrier()
    pltpu.sync_copy(acc.at[pl.ds(base, RPT)], out_hbm.at[c, pl.ds(base, RPT)])


def _deg_call(dst2d):
    k = pl.kernel(
        _deg_body,
        out_type=jax.ShapeDtypeStruct((2, R, 16), F32),
        mesh=_mesh(),
        scratch_types=[
            pltpu.VMEM((NCH // 32, 128), I32),
            pltpu.VMEM((128, 16), F32),
            pltpu.VMEM_SHARED((R, 16), F32),
            pltpu.SemaphoreType.DMA,
        ],
    )
    return k(dst2d)


def _make_agg_body(npc, per_sc_table):
    def body(table_hbm, src_hbm, dst_hbm, out_hbm, srcb, dstb, rows, acc, gsem, ssem):
        c = lax.axis_index("c")
        s = lax.axis_index("s")
        if per_sc_table:
            cb = s * npc
            pltpu.sync_copy(src_hbm.at[c, pl.ds(cb, npc)], srcb)
        else:
            cb = (s * 2 + c) * npc
            pltpu.sync_copy(src_hbm.at[pl.ds(cb, npc)], srcb)
        pltpu.sync_copy(dst_hbm.at[pl.ds(cb, npc)], dstb)

        zero16 = jnp.zeros((16,), F32)

        def zr(i, carry):
            for k in range(8):
                rows[i, pl.ds(k * 16, 16)] = zero16
            return carry

        lax.fori_loop(0, 128, zr, 0)
        base = s * RPT
        for k in range(RPT // 128):
            pltpu.sync_copy(rows, acc.at[pl.ds(base + k * 128, 128)])
        plsc.subcore_barrier()

        def chunk(j, carry):
            pltpu.async_copy(table_hbm.at[srcb.at[j]], rows, gsem).wait()
            pltpu.async_copy(rows, acc.at[dstb.at[j]], ssem, add=True).wait()
            return carry

        lax.fori_loop(0, npc, chunk, 0)
        plsc.subcore_barrier()
        pltpu.sync_copy(acc.at[pl.ds(base, RPT)], out_hbm.at[c, pl.ds(base, RPT)])

    return body


def _agg_call(table, src, dst, npc, per_sc_table):
    k = pl.kernel(
        _make_agg_body(npc, per_sc_table),
        out_type=jax.ShapeDtypeStruct((2, R, H3), F32),
        mesh=_mesh(),
        scratch_types=[
            pltpu.VMEM((npc, 128), I32),
            pltpu.VMEM((npc, 128), I32),
            pltpu.VMEM((128, H3), F32),
            pltpu.VMEM_SHARED((R, H3), F32),
            pltpu.SemaphoreType.DMA,
            pltpu.SemaphoreType.DMA,
        ],
    )
    return k(table, src, dst)


def _dinv_body(degp_ref, out_ref):
    deg = degp_ref[0, :, 0:1] + degp_ref[1, :, 0:1] + 1.0
    row = lax.broadcasted_iota(I32, (R, 1), 0)
    out_ref[...] = jnp.where(row < BN, 1.0 / jnp.sqrt(deg), 0.0)


def _dinv_call(degp):
    return pl.pallas_call(
        _dinv_body,
        out_shape=jax.ShapeDtypeStruct((R, 1), F32),
    )(degp)


def _outer9(v):
    cols = []
    for a in range(3):
        for b in range(3):
            cols.append(v[:, a:a + 1] * v[:, b:b + 1])
    return jnp.concatenate(cols, axis=1)


def _enc_body(pc_ref, dinv_ref, W_ref, b_ref, out_ref):
    pc = pc_ref[0]
    pc12 = jnp.concatenate([pc, _outer9(pc)], axis=1)
    zpad = jnp.zeros((12, 12), F32)
    pc12p = jnp.concatenate([zpad, pc12, zpad], axis=0)
    S = pc12p[0:N]
    for d in range(1, 25):
        S = S + pc12p[d:d + N]
    irow = lax.broadcasted_iota(I32, (N, 1), 0)
    cnt = (jnp.minimum(N, irow + 13) - jnp.maximum(0, irow - 12)).astype(F32)
    S1 = S[:, :3]
    S2 = S[:, 3:]
    m = S1 / cnt
    cov = (S2 - cnt * _outer9(m)) / 23.0
    feat = jnp.concatenate([pc, cov], axis=1)
    x1 = _selu(lax.dot_general(feat, W_ref[...], (((1,), (0,)), ((), ())),
                               preferred_element_type=F32) + b_ref[...])
    out_ref[0] = x1 * dinv_ref[0]


def _enc_call(input_data, dinvB, We1, be1):
    return pl.pallas_call(
        _enc_body,
        grid=(B,),
        in_specs=[
            pl.BlockSpec((1, N, 3), lambda b: (b, 0, 0)),
            pl.BlockSpec((1, N, 1), lambda b: (b, 0, 0)),
            pl.BlockSpec((12, H3), lambda b: (0, 0)),
            pl.BlockSpec((1, H3), lambda b: (0, 0)),
        ],
        out_specs=pl.BlockSpec((1, N, H3), lambda b: (b, 0, 0)),
        out_shape=jax.ShapeDtypeStruct((B, N, H3), F32),
    )(input_data, dinvB, We1, be1)


_MB = 512


def _mid_body(agg_ref, u1_ref, dinv_ref, W_ref, b_ref, out_ref):
    dinv = dinv_ref[...]
    h = (agg_ref[0] + agg_ref[1] + u1_ref[...]) * dinv
    o = _selu(lax.dot_general(h, W_ref[...], (((1,), (0,)), ((), ())),
                              preferred_element_type=F32) + b_ref[...])
    u2 = o * dinv
    out_ref[0] = u2[:, :H3]
    out_ref[1] = u2[:, H3:]


def _mid_call(agg1, u1p, dinv_col, Wg1, bg1):
    return pl.pallas_call(
        _mid_body,
        grid=(R // _MB,),
        in_specs=[
            pl.BlockSpec((2, _MB, H3), lambda r: (0, r, 0)),
            pl.BlockSpec((_MB, H3), lambda r: (r, 0)),
            pl.BlockSpec((_MB, 1), lambda r: (r, 0)),
            pl.BlockSpec((H3, H1), lambda r: (0, 0)),
            pl.BlockSpec((1, H1), lambda r: (0, 0)),
        ],
        out_specs=pl.BlockSpec((2, _MB, H3), lambda r: (0, r, 0)),
        out_shape=jax.ShapeDtypeStruct((2, R, H3), F32),
    )(agg1, u1p, dinv_col, Wg1, bg1)


def _fin_body(agg_ref, u2_ref, dinv_ref, Wg2_ref, bg2_ref, We2_ref, be2_ref,
              Wd1_ref, bd1_ref, Wd2_ref, bd2_ref, grid_ref, out_ref):
    dinv = dinv_ref[0]
    ha = (agg_ref[0, 0] + u2_ref[0, 0]) * dinv
    hb = (agg_ref[1, 0] + u2_ref[1, 0]) * dinv
    Wg2 = Wg2_ref[...]
    o2 = _selu(lax.dot_general(ha, Wg2[:H3], (((1,), (0,)), ((), ())),
                               preferred_element_type=F32)
               + lax.dot_general(hb, Wg2[H3:], (((1,), (0,)), ((), ())),
                                 preferred_element_type=F32)
               + bg2_ref[...])
    p = jnp.max(o2, axis=0, keepdims=True)
    xe = _selu(lax.dot_general(p, We2_ref[...], (((1,), (0,)), ((), ())),
                               preferred_element_type=F32) + be2_ref[...])
    Wd1 = Wd1_ref[...]
    Wd2 = Wd2_ref[...]
    c1 = lax.dot_general(xe, Wd1[:H1], (((1,), (0,)), ((), ())),
                         preferred_element_type=F32) + bd1_ref[...]
    gy = lax.dot_general(grid_ref[...], Wd1[H1:], (((1,), (0,)), ((), ())),
                         preferred_element_type=F32)
    k = _selu(c1 + gy)
    c2 = lax.dot_general(xe, Wd2[:H1], (((1,), (0,)), ((), ())),
                         preferred_element_type=F32) + bd2_ref[...]
    out_ref[0] = _selu(c2 + lax.dot_general(k, Wd2[H1:], (((1,), (0,)), ((), ())),
                                            preferred_element_type=F32))


def _fin_call(agg2r, u2r, dinvB, Wg2, bg2, We2, be2, Wd1, bd1, Wd2, bd2, grid):
    return pl.pallas_call(
        _fin_body,
        grid=(B,),
        in_specs=[
            pl.BlockSpec((2, 1, N, H3), lambda b: (0, b, 0, 0)),
            pl.BlockSpec((2, 1, N, H3), lambda b: (0, b, 0, 0)),
            pl.BlockSpec((1, N, 1), lambda b: (b, 0, 0)),
            pl.BlockSpec((H1, H1), lambda b: (0, 0)),
            pl.BlockSpec((1, H1), lambda b: (0, 0)),
            pl.BlockSpec((H1, H1), lambda b: (0, 0)),
            pl.BlockSpec((1, H1), lambda b: (0, 0)),
            pl.BlockSpec((H1 + 2, 3), lambda b: (0, 0)),
            pl.BlockSpec((1, 3), lambda b: (0, 0)),
            pl.BlockSpec((H1 + 3, 3), lambda b: (0, 0)),
            pl.BlockSpec((1, 3), lambda b: (0, 0)),
            pl.BlockSpec((N, 2), lambda b: (0, 0)),
        ],
        out_specs=pl.BlockSpec((1, N, 3), lambda b: (b, 0, 0)),
        out_shape=jax.ShapeDtypeStruct((B, N, 3), F32),
    )(agg2r, u2r, dinvB, Wg2, bg2, We2, be2, Wd1, bd1, Wd2, bd2, grid)


def _fold_grid():
    nx = int(1 + (N * 120 / 60) ** 0.5)
    ny = int(1 + (N * 60 / 120) ** 0.5)
    xg = np.linspace(1.0, 120.0, nx)
    yg = np.linspace(1.0, 60.0, ny)
    g = np.stack([np.repeat(xg, ny), np.tile(yg, nx)], axis=1)[:N]
    return jnp.asarray(g, F32)


def kernel(input_data, knn, We1, be1, Wg1, bg1, Wg2, bg2, We2, be2, Wd1, bd1, Wd2, bd2):
    pad = EP - E
    src_p = jnp.concatenate([knn[0], jnp.zeros((pad,), I32)])
    dst_p = jnp.concatenate([knn[1], jnp.full((pad,), BN, I32)])
    src2d = src_p.reshape(NCH, 128)
    dst2d = dst_p.reshape(NCH, 128)
    srcK5 = jnp.stack([src2d, src2d + R])

    degp = _deg_call(dst2d)
    dinv_col = _dinv_call(degp)
    dinvB = dinv_col[:BN].reshape(B, N, 1)

    u1_3d = _enc_call(input_data, dinvB, We1, be1.reshape(1, H3))
    u1p = jnp.concatenate([u1_3d.reshape(BN, H3), jnp.zeros((R - BN, H3), F32)])

    agg1 = _agg_call(u1p, src2d, dst2d, NCH // 32, per_sc_table=False)
    u2 = _mid_call(agg1, u1p, dinv_col, Wg1, bg1.reshape(1, H1))

    agg2 = _agg_call(u2.reshape(2 * R, H3), srcK5, dst2d, NCH // 16,
                     per_sc_table=True)
    agg2r = agg2[:, :BN].reshape(2, B, N, H3)
    u2r = u2[:, :BN].reshape(2, B, N, H3)

    return _fin_call(agg2r, u2r, dinvB, Wg2, bg2.reshape(1, H1), We2,
                     be2.reshape(1, H1), Wd1, bd1.reshape(1, 3), Wd2,
                     bd2.reshape(1, 3), _fold_grid())


# speedup vs baseline: 8.1990x; 1.0825x over previous
import functools

import jax
import jax.numpy as jnp
import numpy as np
from jax import lax
from jax.experimental import pallas as pl
from jax.experimental.pallas import tpu as pltpu
from jax.experimental.pallas import tpu_sc as plsc

F32 = jnp.float32
I32 = jnp.int32

H1 = 256
H3 = 128
B = 4
N = 2500
E = 160000
BN = B * N
R = 10240
EP = 163840
NCH = EP // 128
RPT = R // 16
RA = 10112
RAT = RA // 16

_SELU_A = 1.6732632423543772
_SELU_S = 1.0507009873554805


def _selu(x):
    return _SELU_S * jnp.where(x > 0, x, _SELU_A * (jnp.exp(x) - 1.0))


def _mesh():
    return plsc.VectorSubcoreMesh(core_axis_name="c", subcore_axis_name="s")


def _deg_body(dst_hbm, out_hbm, dstb, onesb, acc, sem):
    c = lax.axis_index("c")
    s = lax.axis_index("s")
    wid = s * 2 + c
    npc = NCH // 32
    pltpu.sync_copy(dst_hbm.at[pl.ds(wid * npc, npc)], dstb)
    zero16 = jnp.zeros((16,), F32)
    ones16 = jnp.ones((16,), F32)

    def zbody(i, carry):
        onesb[i, pl.ds(0, 16)] = zero16
        return carry

    lax.fori_loop(0, 128, zbody, 0)
    base = s * RPT
    for k in range(RPT // 128):
        pltpu.sync_copy(onesb, acc.at[pl.ds(base + k * 128, 128)])

    def obody(i, carry):
        onesb[i, pl.ds(0, 16)] = ones16
        return carry

    lax.fori_loop(0, 128, obody, 0)
    plsc.subcore_barrier()

    def fire(j, carry):
        pltpu.async_copy(onesb, acc.at[dstb.at[j]], sem, add=True)
        return carry

    lax.fori_loop(0, npc, fire, 0)

    def drain(j, carry):
        pltpu.make_async_copy(onesb, acc.at[dstb.at[j]], sem).wait()
        return carry

    lax.fori_loop(0, npc, drain, 0)
    plsc.subcore_barrier()
    pltpu.sync_copy(acc.at[pl.ds(base, RPT)], out_hbm.at[c, pl.ds(base, RPT)])


def _deg_call(dst2d):
    k = pl.kernel(
        _deg_body,
        out_type=jax.ShapeDtypeStruct((2, R, 16), F32),
        mesh=_mesh(),
        scratch_types=[
            pltpu.VMEM((NCH // 32, 128), I32),
            pltpu.VMEM((128, 16), F32),
            pltpu.VMEM_SHARED((R, 16), F32),
            pltpu.SemaphoreType.DMA,
        ],
    )
    return k(dst2d)


_NBUF = 2
_GD = 1


def _make_agg_body(npc, per_sc_table, npass):
    npch = npc // npass
    assert npch % _NBUF == 0

    def body(table_hbm, src_hbm, dst_hbm, out_hbm, srcb, dstb,
             rows0, rows1, acc, gsem, ssem):
        bufs = [rows0, rows1]
        c = lax.axis_index("c")
        s = lax.axis_index("s")

        zero16 = jnp.zeros((16,), F32)

        def zr(i, carry):
            for k in range(8):
                rows0[i, pl.ds(k * 16, 16)] = zero16
            return carry

        lax.fori_loop(0, 128, zr, 0)
        base = s * RAT
        for k in range(RAT // 128):
            pltpu.sync_copy(rows0, acc.at[pl.ds(base + k * 128, 128)])
        pltpu.sync_copy(rows0.at[pl.ds(0, RAT % 128)],
                        acc.at[pl.ds(base + (RAT // 128) * 128, RAT % 128)])
        plsc.subcore_barrier()

        def run_pass(cb):
            if per_sc_table:
                pltpu.sync_copy(src_hbm.at[c, pl.ds(cb, npch)], srcb)
            else:
                pltpu.sync_copy(src_hbm.at[pl.ds(cb, npch)], srcb)
            pltpu.sync_copy(dst_hbm.at[pl.ds(cb, npch)], dstb)
            for d in range(_GD):
                pltpu.async_copy(table_hbm.at[srcb.at[d]], bufs[d], gsem)

            def step(j2, carry):
                for jj in range(_NBUF):
                    j = j2 * _NBUF + jj
                    b = bufs[jj]
                    pltpu.make_async_copy(table_hbm.at[srcb.at[j]], b, gsem).wait()
                    pltpu.async_copy(b, acc.at[dstb.at[j]], ssem, add=True)
                    jn = j + _GD
                    bn = bufs[(jj + _GD) % _NBUF]

                    @pl.when(jn - _NBUF >= 0)
                    def _():
                        pltpu.make_async_copy(
                            bn, acc.at[dstb.at[jn - _NBUF]], ssem).wait()

                    @pl.when(jn < npch)
                    def _():
                        pltpu.async_copy(table_hbm.at[srcb.at[jn]], bn, gsem)
                return carry

            lax.fori_loop(0, npch // _NBUF, step, 0)
            for t in range(_NBUF - _GD):
                pltpu.make_async_copy(
                    bufs[0], acc.at[dstb.at[npch - (_NBUF - _GD) + t]], ssem).wait()

        if per_sc_table:
            base_cb = s * npc
        else:
            base_cb = (s * 2 + c) * npc
        for p in range(npass):
            run_pass(base_cb + p * npch)
        plsc.subcore_barrier()
        pltpu.sync_copy(acc.at[pl.ds(base, RAT)], out_hbm.at[c, pl.ds(base, RAT)])

    return body


def _agg_call(table, src, dst, npc, per_sc_table, npass):
    k = pl.kernel(
        _make_agg_body(npc, per_sc_table, npass),
        out_type=jax.ShapeDtypeStruct((2, R, H3), F32),
        mesh=_mesh(),
        scratch_types=[
            pltpu.VMEM((npc // npass, 128), I32),
            pltpu.VMEM((npc // npass, 128), I32),
            pltpu.VMEM((128, H3), F32),
            pltpu.VMEM((128, H3), F32),
            pltpu.VMEM_SHARED((RA, H3), F32),
            pltpu.SemaphoreType.DMA,
            pltpu.SemaphoreType.DMA,
        ],
    )
    return k(table, src, dst)


def _dinv_body(degp_ref, out_ref):
    deg = degp_ref[0, :, 0:1] + degp_ref[1, :, 0:1] + 1.0
    row = lax.broadcasted_iota(I32, (R, 1), 0)
    out_ref[...] = jnp.where(row < BN, 1.0 / jnp.sqrt(deg), 0.0)


def _dinv_call(degp):
    return pl.pallas_call(
        _dinv_body,
        out_shape=jax.ShapeDtypeStruct((R, 1), F32),
    )(degp)


def _outer9(v):
    cols = []
    for a in range(3):
        for b in range(3):
            cols.append(v[:, a:a + 1] * v[:, b:b + 1])
    return jnp.concatenate(cols, axis=1)


def _enc_body(pc_ref, dinv_ref, W_ref, b_ref, out_ref):
    pc = pc_ref[0]
    pc12 = jnp.concatenate([pc, _outer9(pc)], axis=1)
    zpad = jnp.zeros((12, 12), F32)
    pc12p = jnp.concatenate([zpad, pc12, zpad], axis=0)
    S = pc12p[0:N]
    for d in range(1, 25):
        S = S + pc12p[d:d + N]
    irow = lax.broadcasted_iota(I32, (N, 1), 0)
    cnt = (jnp.minimum(N, irow + 13) - jnp.maximum(0, irow - 12)).astype(F32)
    S1 = S[:, :3]
    S2 = S[:, 3:]
    m = S1 / cnt
    cov = (S2 - cnt * _outer9(m)) / 23.0
    feat = jnp.concatenate([pc, cov], axis=1)
    x1 = _selu(lax.dot_general(feat, W_ref[...], (((1,), (0,)), ((), ())),
                               preferred_element_type=F32) + b_ref[...])
    out_ref[0] = x1 * dinv_ref[0]


def _enc_call(input_data, dinvB, We1, be1):
    return pl.pallas_call(
        _enc_body,
        grid=(B,),
        in_specs=[
            pl.BlockSpec((1, N, 3), lambda b: (b, 0, 0)),
            pl.BlockSpec((1, N, 1), lambda b: (b, 0, 0)),
            pl.BlockSpec((12, H3), lambda b: (0, 0)),
            pl.BlockSpec((1, H3), lambda b: (0, 0)),
        ],
        out_specs=pl.BlockSpec((1, N, H3), lambda b: (b, 0, 0)),
        out_shape=jax.ShapeDtypeStruct((B, N, H3), F32),
    )(input_data, dinvB, We1, be1)


_MB = 512


def _mid_body(agg_ref, u1_ref, dinv_ref, W_ref, b_ref, out_ref):
    dinv = dinv_ref[...]
    h = (agg_ref[0] + agg_ref[1] + u1_ref[...]) * dinv
    o = _selu(lax.dot_general(h, W_ref[...], (((1,), (0,)), ((), ())),
                              preferred_element_type=F32) + b_ref[...])
    u2 = o * dinv
    out_ref[0] = u2[:, :H3]
    out_ref[1] = u2[:, H3:]


def _mid_call(agg1, u1p, dinv_col, Wg1, bg1):
    return pl.pallas_call(
        _mid_body,
        grid=(R // _MB,),
        in_specs=[
            pl.BlockSpec((2, _MB, H3), lambda r: (0, r, 0)),
            pl.BlockSpec((_MB, H3), lambda r: (r, 0)),
            pl.BlockSpec((_MB, 1), lambda r: (r, 0)),
            pl.BlockSpec((H3, H1), lambda r: (0, 0)),
            pl.BlockSpec((1, H1), lambda r: (0, 0)),
        ],
        out_specs=pl.BlockSpec((2, _MB, H3), lambda r: (0, r, 0)),
        out_shape=jax.ShapeDtypeStruct((2, R, H3), F32),
    )(agg1, u1p, dinv_col, Wg1, bg1)


def _fin_body(agg_ref, u2_ref, dinv_ref, Wg2_ref, bg2_ref, We2_ref, be2_ref,
              Wd1_ref, bd1_ref, Wd2_ref, bd2_ref, grid_ref, out_ref):
    dinv = dinv_ref[0]
    ha = (agg_ref[0, 0] + u2_ref[0, 0]) * dinv
    hb = (agg_ref[1, 0] + u2_ref[1, 0]) * dinv
    Wg2 = Wg2_ref[...]
    o2 = _selu(lax.dot_general(ha, Wg2[:H3], (((1,), (0,)), ((), ())),
                               preferred_element_type=F32)
               + lax.dot_general(hb, Wg2[H3:], (((1,), (0,)), ((), ())),
                                 preferred_element_type=F32)
               + bg2_ref[...])
    p = jnp.max(o2, axis=0, keepdims=True)
    xe = _selu(lax.dot_general(p, We2_ref[...], (((1,), (0,)), ((), ())),
                               preferred_element_type=F32) + be2_ref[...])
    Wd1 = Wd1_ref[...]
    Wd2 = Wd2_ref[...]
    c1 = lax.dot_general(xe, Wd1[:H1], (((1,), (0,)), ((), ())),
                         preferred_element_type=F32) + bd1_ref[...]
    gy = lax.dot_general(grid_ref[...], Wd1[H1:], (((1,), (0,)), ((), ())),
                         preferred_element_type=F32)
    k = _selu(c1 + gy)
    c2 = lax.dot_general(xe, Wd2[:H1], (((1,), (0,)), ((), ())),
                         preferred_element_type=F32) + bd2_ref[...]
    out_ref[0] = _selu(c2 + lax.dot_general(k, Wd2[H1:], (((1,), (0,)), ((), ())),
                                            preferred_element_type=F32))


def _fin_call(agg2r, u2r, dinvB, Wg2, bg2, We2, be2, Wd1, bd1, Wd2, bd2, grid):
    return pl.pallas_call(
        _fin_body,
        grid=(B,),
        in_specs=[
            pl.BlockSpec((2, 1, N, H3), lambda b: (0, b, 0, 0)),
            pl.BlockSpec((2, 1, N, H3), lambda b: (0, b, 0, 0)),
            pl.BlockSpec((1, N, 1), lambda b: (b, 0, 0)),
            pl.BlockSpec((H1, H1), lambda b: (0, 0)),
            pl.BlockSpec((1, H1), lambda b: (0, 0)),
            pl.BlockSpec((H1, H1), lambda b: (0, 0)),
            pl.BlockSpec((1, H1), lambda b: (0, 0)),
            pl.BlockSpec((H1 + 2, 3), lambda b: (0, 0)),
            pl.BlockSpec((1, 3), lambda b: (0, 0)),
            pl.BlockSpec((H1 + 3, 3), lambda b: (0, 0)),
            pl.BlockSpec((1, 3), lambda b: (0, 0)),
            pl.BlockSpec((N, 2), lambda b: (0, 0)),
        ],
        out_specs=pl.BlockSpec((1, N, 3), lambda b: (b, 0, 0)),
        out_shape=jax.ShapeDtypeStruct((B, N, 3), F32),
    )(agg2r, u2r, dinvB, Wg2, bg2, We2, be2, Wd1, bd1, Wd2, bd2, grid)


def _fold_grid():
    nx = int(1 + (N * 120 / 60) ** 0.5)
    ny = int(1 + (N * 60 / 120) ** 0.5)
    xg = np.linspace(1.0, 120.0, nx)
    yg = np.linspace(1.0, 60.0, ny)
    g = np.stack([np.repeat(xg, ny), np.tile(yg, nx)], axis=1)[:N]
    return jnp.asarray(g, F32)


def kernel(input_data, knn, We1, be1, Wg1, bg1, Wg2, bg2, We2, be2, Wd1, bd1, Wd2, bd2):
    pad = EP - E
    src_p = jnp.concatenate([knn[0], jnp.zeros((pad,), I32)])
    dst_p = jnp.concatenate([knn[1], jnp.full((pad,), BN, I32)])
    src2d = src_p.reshape(NCH, 128)
    dst2d = dst_p.reshape(NCH, 128)
    srcK5 = jnp.stack([src2d, src2d + R])

    degp = _deg_call(dst2d)
    dinv_col = _dinv_call(degp)
    dinvB = dinv_col[:BN].reshape(B, N, 1)

    u1_3d = _enc_call(input_data, dinvB, We1, be1.reshape(1, H3))
    u1p = jnp.concatenate([u1_3d.reshape(BN, H3), jnp.zeros((R - BN, H3), F32)])

    agg1 = _agg_call(u1p, src2d, dst2d, NCH // 32, per_sc_table=False, npass=1)
    u2 = _mid_call(agg1, u1p, dinv_col, Wg1, bg1.reshape(1, H1))

    agg2 = _agg_call(u2.reshape(2 * R, H3), srcK5, dst2d, NCH // 16,
                     per_sc_table=True, npass=2)
    agg2r = agg2[:, :BN].reshape(2, B, N, H3)
    u2r = u2[:, :BN].reshape(2, B, N, H3)

    return _fin_call(agg2r, u2r, dinvB, Wg2, bg2.reshape(1, H1), We2,
                     be2.reshape(1, H1), Wd1, bd1.reshape(1, 3), Wd2,
                     bd2.reshape(1, 3), _fold_grid())


# speedup vs baseline: 8.5038x; 1.0372x over previous
import functools

import jax
import jax.numpy as jnp
import numpy as np
from jax import lax
from jax.experimental import pallas as pl
from jax.experimental.pallas import tpu as pltpu
from jax.experimental.pallas import tpu_sc as plsc

F32 = jnp.float32
I32 = jnp.int32

H1 = 256
H3 = 128
B = 4
N = 2500
E = 160000
BN = B * N
R = 10240
EP = 163840
NCH = EP // 128
RPT = R // 16
RA = 10112
RAT = RA // 16

_SELU_A = 1.6732632423543772
_SELU_S = 1.0507009873554805


def _selu(x):
    return _SELU_S * jnp.where(x > 0, x, _SELU_A * (jnp.exp(x) - 1.0))


def _mesh():
    return plsc.VectorSubcoreMesh(core_axis_name="c", subcore_axis_name="s")


def _deg_body(dst_hbm, out_hbm, dstb, onesb, acc, sem):
    c = lax.axis_index("c")
    s = lax.axis_index("s")
    wid = s * 2 + c
    npc = NCH // 32
    pltpu.sync_copy(dst_hbm.at[pl.ds(wid * npc, npc)], dstb)
    zero16 = jnp.zeros((16,), F32)
    ones16 = jnp.ones((16,), F32)

    def zbody(i, carry):
        onesb[i, pl.ds(0, 16)] = zero16
        return carry

    lax.fori_loop(0, 128, zbody, 0)
    base = s * RPT
    for k in range(RPT // 128):
        pltpu.sync_copy(onesb, acc.at[pl.ds(base + k * 128, 128)])

    def obody(i, carry):
        onesb[i, pl.ds(0, 16)] = ones16
        return carry

    lax.fori_loop(0, 128, obody, 0)
    plsc.subcore_barrier()

    def fire(j, carry):
        pltpu.async_copy(onesb, acc.at[dstb.at[j]], sem, add=True)
        return carry

    lax.fori_loop(0, npc, fire, 0)

    def drain(j, carry):
        pltpu.make_async_copy(onesb, acc.at[dstb.at[j]], sem).wait()
        return carry

    lax.fori_loop(0, npc, drain, 0)
    plsc.subcore_barrier()
    pltpu.sync_copy(acc.at[pl.ds(base, RPT)], out_hbm.at[c, pl.ds(base, RPT)])


def _deg_call(dst2d):
    k = pl.kernel(
        _deg_body,
        out_type=jax.ShapeDtypeStruct((2, R, 16), F32),
        mesh=_mesh(),
        scratch_types=[
            pltpu.VMEM((NCH // 32, 128), I32),
            pltpu.VMEM((128, 16), F32),
            pltpu.VMEM_SHARED((R, 16), F32),
            pltpu.SemaphoreType.DMA,
        ],
    )
    return k(dst2d)


_NBUF = 4
_GD = 2
CH = 64
NCC = EP // CH


def _make_agg_body(npc, per_sc_table, npass):
    npch = npc // npass
    assert npch % _NBUF == 0

    def body(table_hbm, src_hbm, dst_hbm, out_hbm, srcb, dstb,
             rows0, rows1, rows2, rows3, acc, gsem, ssem):
        bufs = [rows0, rows1, rows2, rows3]
        c = lax.axis_index("c")
        s = lax.axis_index("s")

        zero16 = jnp.zeros((16,), F32)

        def zr(i, carry):
            for k in range(8):
                rows0[i, pl.ds(k * 16, 16)] = zero16
            return carry

        lax.fori_loop(0, CH, zr, 0)
        base = s * RAT
        for k in range(RAT // CH):
            pltpu.sync_copy(rows0, acc.at[pl.ds(base + k * CH, CH)])
        if RAT % CH:
            pltpu.sync_copy(rows0.at[pl.ds(0, RAT % CH)],
                            acc.at[pl.ds(base + (RAT // CH) * CH, RAT % CH)])
        plsc.subcore_barrier()

        def run_pass(cb):
            if per_sc_table:
                pltpu.sync_copy(src_hbm.at[c, pl.ds(cb, npch)], srcb)
            else:
                pltpu.sync_copy(src_hbm.at[pl.ds(cb, npch)], srcb)
            pltpu.sync_copy(dst_hbm.at[pl.ds(cb, npch)], dstb)
            for d in range(_GD):
                pltpu.async_copy(table_hbm.at[srcb.at[d]], bufs[d], gsem)

            def step(j2, carry):
                for jj in range(_NBUF):
                    j = j2 * _NBUF + jj
                    b = bufs[jj]
                    pltpu.make_async_copy(table_hbm.at[srcb.at[j]], b, gsem).wait()
                    pltpu.async_copy(b, acc.at[dstb.at[j]], ssem, add=True)
                    jn = j + _GD
                    bn = bufs[(jj + _GD) % _NBUF]

                    @pl.when(jn - _NBUF >= 0)
                    def _():
                        pltpu.make_async_copy(
                            bn, acc.at[dstb.at[jn - _NBUF]], ssem).wait()

                    @pl.when(jn < npch)
                    def _():
                        pltpu.async_copy(table_hbm.at[srcb.at[jn]], bn, gsem)
                return carry

            lax.fori_loop(0, npch // _NBUF, step, 0)
            for t in range(_NBUF - _GD):
                pltpu.make_async_copy(
                    bufs[0], acc.at[dstb.at[npch - (_NBUF - _GD) + t]], ssem).wait()

        if per_sc_table:
            base_cb = s * npc
        else:
            base_cb = (s * 2 + c) * npc
        for p in range(npass):
            run_pass(base_cb + p * npch)
        plsc.subcore_barrier()
        pltpu.sync_copy(acc.at[pl.ds(base, RAT)], out_hbm.at[c, pl.ds(base, RAT)])

    return body


def _agg_call(table, src, dst, npc, per_sc_table, npass):
    k = pl.kernel(
        _make_agg_body(npc, per_sc_table, npass),
        out_type=jax.ShapeDtypeStruct((2, R, H3), F32),
        mesh=_mesh(),
        scratch_types=[
            pltpu.VMEM((npc // npass, CH), I32),
            pltpu.VMEM((npc // npass, CH), I32),
            pltpu.VMEM((CH, H3), F32),
            pltpu.VMEM((CH, H3), F32),
            pltpu.VMEM((CH, H3), F32),
            pltpu.VMEM((CH, H3), F32),
            pltpu.VMEM_SHARED((RA, H3), F32),
            pltpu.SemaphoreType.DMA,
            pltpu.SemaphoreType.DMA,
        ],
    )
    return k(table, src, dst)


def _dinv_body(degp_ref, out_ref):
    deg = degp_ref[0, :, 0:1] + degp_ref[1, :, 0:1] + 1.0
    row = lax.broadcasted_iota(I32, (R, 1), 0)
    out_ref[...] = jnp.where(row < BN, 1.0 / jnp.sqrt(deg), 0.0)


def _dinv_call(degp):
    return pl.pallas_call(
        _dinv_body,
        out_shape=jax.ShapeDtypeStruct((R, 1), F32),
    )(degp)


def _outer9(v):
    cols = []
    for a in range(3):
        for b in range(3):
            cols.append(v[:, a:a + 1] * v[:, b:b + 1])
    return jnp.concatenate(cols, axis=1)


def _enc_body(pc_ref, dinv_ref, W_ref, b_ref, out_ref):
    pc = pc_ref[0]
    pc12 = jnp.concatenate([pc, _outer9(pc)], axis=1)
    zpad = jnp.zeros((12, 12), F32)
    pc12p = jnp.concatenate([zpad, pc12, zpad], axis=0)
    S = pc12p[0:N]
    for d in range(1, 25):
        S = S + pc12p[d:d + N]
    irow = lax.broadcasted_iota(I32, (N, 1), 0)
    cnt = (jnp.minimum(N, irow + 13) - jnp.maximum(0, irow - 12)).astype(F32)
    S1 = S[:, :3]
    S2 = S[:, 3:]
    m = S1 / cnt
    cov = (S2 - cnt * _outer9(m)) / 23.0
    feat = jnp.concatenate([pc, cov], axis=1)
    x1 = _selu(lax.dot_general(feat, W_ref[...], (((1,), (0,)), ((), ())),
                               preferred_element_type=F32) + b_ref[...])
    out_ref[0] = x1 * dinv_ref[0]


def _enc_call(input_data, dinvB, We1, be1):
    return pl.pallas_call(
        _enc_body,
        grid=(B,),
        in_specs=[
            pl.BlockSpec((1, N, 3), lambda b: (b, 0, 0)),
            pl.BlockSpec((1, N, 1), lambda b: (b, 0, 0)),
            pl.BlockSpec((12, H3), lambda b: (0, 0)),
            pl.BlockSpec((1, H3), lambda b: (0, 0)),
        ],
        out_specs=pl.BlockSpec((1, N, H3), lambda b: (b, 0, 0)),
        out_shape=jax.ShapeDtypeStruct((B, N, H3), F32),
    )(input_data, dinvB, We1, be1)


_MB = 512


def _mid_body(agg_ref, u1_ref, dinv_ref, W_ref, b_ref, out_ref):
    dinv = dinv_ref[...]
    h = (agg_ref[0] + agg_ref[1] + u1_ref[...]) * dinv
    o = _selu(lax.dot_general(h, W_ref[...], (((1,), (0,)), ((), ())),
                              preferred_element_type=F32) + b_ref[...])
    u2 = o * dinv
    out_ref[0] = u2[:, :H3]
    out_ref[1] = u2[:, H3:]


def _mid_call(agg1, u1p, dinv_col, Wg1, bg1):
    return pl.pallas_call(
        _mid_body,
        grid=(R // _MB,),
        in_specs=[
            pl.BlockSpec((2, _MB, H3), lambda r: (0, r, 0)),
            pl.BlockSpec((_MB, H3), lambda r: (r, 0)),
            pl.BlockSpec((_MB, 1), lambda r: (r, 0)),
            pl.BlockSpec((H3, H1), lambda r: (0, 0)),
            pl.BlockSpec((1, H1), lambda r: (0, 0)),
        ],
        out_specs=pl.BlockSpec((2, _MB, H3), lambda r: (0, r, 0)),
        out_shape=jax.ShapeDtypeStruct((2, R, H3), F32),
    )(agg1, u1p, dinv_col, Wg1, bg1)


def _fin_body(agg_ref, u2_ref, dinv_ref, Wg2_ref, bg2_ref, We2_ref, be2_ref,
              Wd1_ref, bd1_ref, Wd2_ref, bd2_ref, grid_ref, out_ref):
    dinv = dinv_ref[0]
    ha = (agg_ref[0, 0] + u2_ref[0, 0]) * dinv
    hb = (agg_ref[1, 0] + u2_ref[1, 0]) * dinv
    Wg2 = Wg2_ref[...]
    o2 = _selu(lax.dot_general(ha, Wg2[:H3], (((1,), (0,)), ((), ())),
                               preferred_element_type=F32)
               + lax.dot_general(hb, Wg2[H3:], (((1,), (0,)), ((), ())),
                                 preferred_element_type=F32)
               + bg2_ref[...])
    p = jnp.max(o2, axis=0, keepdims=True)
    xe = _selu(lax.dot_general(p, We2_ref[...], (((1,), (0,)), ((), ())),
                               preferred_element_type=F32) + be2_ref[...])
    Wd1 = Wd1_ref[...]
    Wd2 = Wd2_ref[...]
    c1 = lax.dot_general(xe, Wd1[:H1], (((1,), (0,)), ((), ())),
                         preferred_element_type=F32) + bd1_ref[...]
    gy = lax.dot_general(grid_ref[...], Wd1[H1:], (((1,), (0,)), ((), ())),
                         preferred_element_type=F32)
    k = _selu(c1 + gy)
    c2 = lax.dot_general(xe, Wd2[:H1], (((1,), (0,)), ((), ())),
                         preferred_element_type=F32) + bd2_ref[...]
    out_ref[0] = _selu(c2 + lax.dot_general(k, Wd2[H1:], (((1,), (0,)), ((), ())),
                                            preferred_element_type=F32))


def _fin_call(agg2r, u2r, dinvB, Wg2, bg2, We2, be2, Wd1, bd1, Wd2, bd2, grid):
    return pl.pallas_call(
        _fin_body,
        grid=(B,),
        in_specs=[
            pl.BlockSpec((2, 1, N, H3), lambda b: (0, b, 0, 0)),
            pl.BlockSpec((2, 1, N, H3), lambda b: (0, b, 0, 0)),
            pl.BlockSpec((1, N, 1), lambda b: (b, 0, 0)),
            pl.BlockSpec((H1, H1), lambda b: (0, 0)),
            pl.BlockSpec((1, H1), lambda b: (0, 0)),
            pl.BlockSpec((H1, H1), lambda b: (0, 0)),
            pl.BlockSpec((1, H1), lambda b: (0, 0)),
            pl.BlockSpec((H1 + 2, 3), lambda b: (0, 0)),
            pl.BlockSpec((1, 3), lambda b: (0, 0)),
            pl.BlockSpec((H1 + 3, 3), lambda b: (0, 0)),
            pl.BlockSpec((1, 3), lambda b: (0, 0)),
            pl.BlockSpec((N, 2), lambda b: (0, 0)),
        ],
        out_specs=pl.BlockSpec((1, N, 3), lambda b: (b, 0, 0)),
        out_shape=jax.ShapeDtypeStruct((B, N, 3), F32),
    )(agg2r, u2r, dinvB, Wg2, bg2, We2, be2, Wd1, bd1, Wd2, bd2, grid)


def _fold_grid():
    nx = int(1 + (N * 120 / 60) ** 0.5)
    ny = int(1 + (N * 60 / 120) ** 0.5)
    xg = np.linspace(1.0, 120.0, nx)
    yg = np.linspace(1.0, 60.0, ny)
    g = np.stack([np.repeat(xg, ny), np.tile(yg, nx)], axis=1)[:N]
    return jnp.asarray(g, F32)


def kernel(input_data, knn, We1, be1, Wg1, bg1, Wg2, bg2, We2, be2, Wd1, bd1, Wd2, bd2):
    pad = EP - E
    src_p = jnp.concatenate([knn[0], jnp.zeros((pad,), I32)])
    dst_p = jnp.concatenate([knn[1], jnp.full((pad,), BN, I32)])
    src2d = src_p.reshape(NCH, 128)
    dst2d = dst_p.reshape(NCH, 128)
    srcA = src_p.reshape(NCC, CH)
    dstA = dst_p.reshape(NCC, CH)
    srcK5 = jnp.stack([srcA, srcA + R])

    degp = _deg_call(dst2d)
    dinv_col = _dinv_call(degp)
    dinvB = dinv_col[:BN].reshape(B, N, 1)

    u1_3d = _enc_call(input_data, dinvB, We1, be1.reshape(1, H3))
    u1p = jnp.concatenate([u1_3d.reshape(BN, H3), jnp.zeros((R - BN, H3), F32)])

    agg1 = _agg_call(u1p, srcA, dstA, NCC // 32, per_sc_table=False, npass=2)
    u2 = _mid_call(agg1, u1p, dinv_col, Wg1, bg1.reshape(1, H1))

    agg2 = _agg_call(u2.reshape(2 * R, H3), srcK5, dstA, NCC // 16,
                     per_sc_table=True, npass=4)
    agg2r = agg2[:, :BN].reshape(2, B, N, H3)
    u2r = u2[:, :BN].reshape(2, B, N, H3)

    return _fin_call(agg2r, u2r, dinvB, Wg2, bg2.reshape(1, H1), We2,
                     be2.reshape(1, H1), Wd1, bd1.reshape(1, 3), Wd2,
                     bd2.reshape(1, 3), _fold_grid())
